# baseline (speedup 1.0000x reference)
import functools

import jax
import jax.numpy as jnp
from jax import lax
from jax.experimental import pallas as pl
from jax.experimental.pallas import tpu as pltpu
from jax.experimental.pallas import tpu_sc as plsc

F = 64
B = 1024
FTS = 128
U1 = 256
U2 = 128
DIM = 32
EMB = 8192

CHUNK = 1024
NCHUNK = EMB // CHUNK

NC = 2
NS = 16
NW = NC * NS
BPW = (F * B) // NW
GCH = 128
NGCH = BPW // GCH


def _encode_body(x_ref, w1_ref, b1_ref, w2_ref, b2_ref, w3_ref, b3_ref,
                 cb_ref, idx_ref):
    x = x_ref[...]
    h = jnp.maximum(
        jnp.dot(x, w1_ref[0], preferred_element_type=jnp.float32)
        + b1_ref[0], 0.0)
    h = jnp.maximum(
        jnp.dot(h, w2_ref[0], preferred_element_type=jnp.float32)
        + b2_ref[0], 0.0)
    z = jnp.maximum(
        jnp.dot(h, w3_ref[0], preferred_element_type=jnp.float32)
        + b3_ref[0], 0.0)
    z_aug = jnp.concatenate(
        [z, jnp.full((B, 1), -1.0, jnp.float32),
         jnp.zeros((B, 7), jnp.float32)], axis=1)
    runbest = jnp.full((B, 128), -jnp.inf, dtype=jnp.float32)
    runtile = jnp.zeros((B, 128), dtype=jnp.float32)
    ntile = CHUNK // 128
    for c in range(NCHUNK):
        cb = cb_ref[:, c * CHUNK:(c + 1) * CHUNK]
        csq2 = 0.5 * jnp.sum(cb * cb, axis=0, keepdims=True)
        cb_aug = jnp.concatenate(
            [cb, csq2, jnp.zeros((7, CHUNK), jnp.float32)], axis=0)
        s2 = jnp.dot(z_aug, cb_aug, preferred_element_type=jnp.float32)
        for t in range(ntile):
            st = s2[:, t * 128:(t + 1) * 128]
            upd = st > runbest
            runbest = jnp.where(upd, st, runbest)
            runtile = jnp.where(upd, float(c * ntile + t), runtile)
    best = jnp.max(runbest, axis=1, keepdims=True)
    lane = lax.broadcasted_iota(jnp.int32, (B, 128), 1).astype(jnp.float32)
    colg = runtile * 128.0 + lane
    cand = jnp.min(jnp.where(runbest >= best, colg, 3.0e7),
                   axis=1, keepdims=True)
    idx_ref[...] = cand.astype(jnp.int32)


def _decode_body(q_ref, w4_ref, b4_ref, w5_ref, b5_ref, w6_ref, b6_ref,
                 o_ref):
    q = q_ref[0][:, :DIM]
    h = jnp.maximum(
        jnp.dot(q, w4_ref[0], preferred_element_type=jnp.float32)
        + b4_ref[0], 0.0)
    h = jnp.maximum(
        jnp.dot(h, w5_ref[0], preferred_element_type=jnp.float32)
        + b5_ref[0], 0.0)
    y = jnp.dot(h, w6_ref[0], preferred_element_type=jnp.float32) + b6_ref[0]
    o_ref[...] = 1.0 / (1.0 + jnp.exp(-y))


RND = 4
SPR = NGCH // RND
RROWS = SPR * GCH


@functools.cache
def _gather_rows_kernel():
    mesh = plsc.VectorSubcoreMesh(core_axis_name="c", subcore_axis_name="s",
                                  num_cores=NC)

    @functools.partial(
        pl.kernel,
        mesh=mesh,
        out_type=jax.ShapeDtypeStruct((F * B, 128), jnp.float32),
        scratch_types=[
            pltpu.VMEM((NGCH, GCH), jnp.int32),
            pltpu.VMEM((RROWS, 128), jnp.float32),
            pltpu.SemaphoreType.DMA,
        ],
    )
    def _gather_rows(table_hbm, idx_hbm, out_hbm, idx_v, rows_v, sem):
        wid = lax.axis_index("s") * NC + lax.axis_index("c")
        pltpu.sync_copy(idx_hbm.at[wid], idx_v)
        for r in range(RND):
            copies = [
                pltpu.async_copy(table_hbm.at[idx_v.at[r * SPR + j]],
                                 rows_v.at[pl.ds(j * GCH, GCH)], sem)
                for j in range(SPR)
            ]
            for cp in copies:
                cp.wait()
            pltpu.sync_copy(
                rows_v, out_hbm.at[pl.ds(wid * BPW + r * RROWS, RROWS)])

    return _gather_rows


def _encode_call(x2d, W1, b1r, W2, b2r, W3, b3r, codebook, interpret=False):
    return pl.pallas_call(
        _encode_body,
        grid=(F,),
        in_specs=[
            pl.BlockSpec((B, FTS), lambda f: (0, f)),
            pl.BlockSpec((1, FTS, U1), lambda f: (f, 0, 0)),
            pl.BlockSpec((1, 1, U1), lambda f: (f, 0, 0)),
            pl.BlockSpec((1, U1, U2), lambda f: (f, 0, 0)),
            pl.BlockSpec((1, 1, U2), lambda f: (f, 0, 0)),
            pl.BlockSpec((1, U2, DIM), lambda f: (f, 0, 0)),
            pl.BlockSpec((1, 1, DIM), lambda f: (f, 0, 0)),
            pl.BlockSpec((DIM, EMB), lambda f: (0, 0)),
        ],
        out_specs=pl.BlockSpec((B, 1), lambda f: (f, 0)),
        out_shape=jax.ShapeDtypeStruct((F * B, 1), jnp.int32),
        compiler_params=pltpu.CompilerParams(
            dimension_semantics=("parallel",)),
        interpret=interpret,
    )(x2d, W1, b1r, W2, b2r, W3, b3r, codebook)


def _decode_call(q3, W4, b4r, W5, b5r, W6, b6r, interpret=False):
    return pl.pallas_call(
        _decode_body,
        grid=(F,),
        in_specs=[
            pl.BlockSpec((1, B, 128), lambda f: (f, 0, 0)),
            pl.BlockSpec((1, DIM, U2), lambda f: (f, 0, 0)),
            pl.BlockSpec((1, 1, U2), lambda f: (f, 0, 0)),
            pl.BlockSpec((1, U2, U1), lambda f: (f, 0, 0)),
            pl.BlockSpec((1, 1, U1), lambda f: (f, 0, 0)),
            pl.BlockSpec((1, U1, FTS), lambda f: (f, 0, 0)),
            pl.BlockSpec((1, 1, FTS), lambda f: (f, 0, 0)),
        ],
        out_specs=pl.BlockSpec((B, FTS), lambda f: (0, f)),
        out_shape=jax.ShapeDtypeStruct((B, F * FTS), jnp.float32),
        compiler_params=pltpu.CompilerParams(
            dimension_semantics=("parallel",)),
        interpret=interpret,
    )(q3, W4, b4r, W5, b5r, W6, b6r)


def kernel(inputs, W1, b1, W2, b2, W3, b3, W4, b4, W5, b5, W6, b6, codebook):
    x2d = inputs.reshape(B, F * FTS)
    idx = _encode_call(x2d, W1, b1.reshape(F, 1, U1), W2, b2.reshape(F, 1, U2),
                       W3, b3.reshape(F, 1, DIM), codebook)
    table = jnp.pad(codebook.T, ((0, 0), (0, 128 - DIM)))
    quant = _gather_rows_kernel()(table, idx.reshape(NW, NGCH, GCH))
    out2d = _decode_call(quant.reshape(F, B, 128),
                         W4, b4.reshape(F, 1, U2), W5, b5.reshape(F, 1, U1),
                         W6, b6.reshape(F, 1, FTS))
    return out2d.reshape(B, F, FTS)


# speedup vs baseline: 1.1470x; 1.1470x over previous
import functools

import jax
import jax.numpy as jnp
from jax import lax
from jax.experimental import pallas as pl
from jax.experimental.pallas import tpu as pltpu
from jax.experimental.pallas import tpu_sc as plsc

F = 64
B = 1024
FTS = 128
U1 = 256
U2 = 128
DIM = 32
EMB = 8192

FG = 8
NG = F // FG

CHUNK = 1024
NCHUNK = EMB // CHUNK

NC = 2
NS = 16
NW = NC * NS
BPW = (F * B) // NW
GCH = 128
NGCH = BPW // GCH


def _encode_body(x_ref, w1_ref, b1_ref, w2_ref, b2_ref, w3_ref, b3_ref,
                 cb_ref, idx_ref):
    ntile = CHUNK // 128
    for i in range(FG):
        x = x_ref[:, i, :]
        h = jnp.maximum(
            jnp.dot(x, w1_ref[i], preferred_element_type=jnp.float32)
            + b1_ref[i], 0.0)
        h = jnp.maximum(
            jnp.dot(h, w2_ref[i], preferred_element_type=jnp.float32)
            + b2_ref[i], 0.0)
        z = jnp.maximum(
            jnp.dot(h, w3_ref[i], preferred_element_type=jnp.float32)
            + b3_ref[i], 0.0)
        z_aug = jnp.concatenate(
            [z, jnp.full((B, 1), -1.0, jnp.float32),
             jnp.zeros((B, 7), jnp.float32)], axis=1)
        runbest = jnp.full((B, 128), -jnp.inf, dtype=jnp.float32)
        runtile = jnp.zeros((B, 128), dtype=jnp.float32)
        for c in range(NCHUNK):
            cb = cb_ref[:, c * CHUNK:(c + 1) * CHUNK]
            csq2 = 0.5 * jnp.sum(cb * cb, axis=0, keepdims=True)
            cb_aug = jnp.concatenate(
                [cb, csq2, jnp.zeros((7, CHUNK), jnp.float32)], axis=0)
            s2 = jnp.dot(z_aug, cb_aug, preferred_element_type=jnp.float32)
            for t in range(ntile):
                st = s2[:, t * 128:(t + 1) * 128]
                upd = st > runbest
                runbest = jnp.where(upd, st, runbest)
                runtile = jnp.where(upd, float(c * ntile + t), runtile)
        best = jnp.max(runbest, axis=1, keepdims=True)
        lane = lax.broadcasted_iota(jnp.int32, (B, 128), 1).astype(jnp.float32)
        colg = runtile * 128.0 + lane
        cand = jnp.min(jnp.where(runbest >= best, colg, 3.0e7),
                       axis=1, keepdims=True)
        idx_ref[i * B:(i + 1) * B, :] = cand.astype(jnp.int32)


def _decode_body(q_ref, w4_ref, b4_ref, w5_ref, b5_ref, w6_ref, b6_ref,
                 o_ref):
    for i in range(FG):
        q = q_ref[i]
        h = jnp.maximum(
            jnp.dot(q, w4_ref[i], preferred_element_type=jnp.float32)
            + b4_ref[i], 0.0)
        h = jnp.maximum(
            jnp.dot(h, w5_ref[i], preferred_element_type=jnp.float32)
            + b5_ref[i], 0.0)
        y = (jnp.dot(h, w6_ref[i], preferred_element_type=jnp.float32)
             + b6_ref[i])
        o_ref[:, i, :] = 1.0 / (1.0 + jnp.exp(-y))


@functools.cache
def _gather_rows_kernel():
    mesh = plsc.VectorSubcoreMesh(core_axis_name="c", subcore_axis_name="s",
                                  num_cores=NC)

    @functools.partial(
        pl.kernel,
        mesh=mesh,
        out_type=jax.ShapeDtypeStruct((F * B, DIM), jnp.float32),
        scratch_types=[
            pltpu.VMEM((NGCH, GCH), jnp.int32),
            pltpu.VMEM((BPW, DIM), jnp.float32),
            pltpu.SemaphoreType.DMA,
        ],
        compiler_params=pltpu.CompilerParams(use_tc_tiling_on_sc=False),
    )
    def _gather_rows(table_hbm, idx_hbm, out_hbm, idx_v, rows_v, sem):
        wid = lax.axis_index("s") * NC + lax.axis_index("c")
        pltpu.sync_copy(idx_hbm.at[wid], idx_v)
        copies = [
            pltpu.async_copy(table_hbm.at[idx_v.at[j]],
                             rows_v.at[pl.ds(j * GCH, GCH)], sem)
            for j in range(NGCH)
        ]
        for cp in copies:
            cp.wait()
        pltpu.sync_copy(rows_v, out_hbm.at[pl.ds(wid * BPW, BPW)])

    return _gather_rows


def _encode_call(x3d, W1, b1r, W2, b2r, W3, b3r, codebook, interpret=False):
    return pl.pallas_call(
        _encode_body,
        grid=(NG,),
        in_specs=[
            pl.BlockSpec((B, FG, FTS), lambda g: (0, g, 0)),
            pl.BlockSpec((FG, FTS, U1), lambda g: (g, 0, 0)),
            pl.BlockSpec((FG, 1, U1), lambda g: (g, 0, 0)),
            pl.BlockSpec((FG, U1, U2), lambda g: (g, 0, 0)),
            pl.BlockSpec((FG, 1, U2), lambda g: (g, 0, 0)),
            pl.BlockSpec((FG, U2, DIM), lambda g: (g, 0, 0)),
            pl.BlockSpec((FG, 1, DIM), lambda g: (g, 0, 0)),
            pl.BlockSpec((DIM, EMB), lambda g: (0, 0)),
        ],
        out_specs=pl.BlockSpec((FG * B, 1), lambda g: (g, 0)),
        out_shape=jax.ShapeDtypeStruct((F * B, 1), jnp.int32),
        compiler_params=pltpu.CompilerParams(
            dimension_semantics=("parallel",)),
        interpret=interpret,
    )(x3d, W1, b1r, W2, b2r, W3, b3r, codebook)


def _decode_call(q3, W4, b4r, W5, b5r, W6, b6r, interpret=False):
    return pl.pallas_call(
        _decode_body,
        grid=(NG,),
        in_specs=[
            pl.BlockSpec((FG, B, DIM), lambda g: (g, 0, 0)),
            pl.BlockSpec((FG, DIM, U2), lambda g: (g, 0, 0)),
            pl.BlockSpec((FG, 1, U2), lambda g: (g, 0, 0)),
            pl.BlockSpec((FG, U2, U1), lambda g: (g, 0, 0)),
            pl.BlockSpec((FG, 1, U1), lambda g: (g, 0, 0)),
            pl.BlockSpec((FG, U1, FTS), lambda g: (g, 0, 0)),
            pl.BlockSpec((FG, 1, FTS), lambda g: (g, 0, 0)),
        ],
        out_specs=pl.BlockSpec((B, FG, FTS), lambda g: (0, g, 0)),
        out_shape=jax.ShapeDtypeStruct((B, F, FTS), jnp.float32),
        compiler_params=pltpu.CompilerParams(
            dimension_semantics=("parallel",)),
        interpret=interpret,
    )(q3, W4, b4r, W5, b5r, W6, b6r)


def kernel(inputs, W1, b1, W2, b2, W3, b3, W4, b4, W5, b5, W6, b6, codebook):
    idx = _encode_call(inputs, W1, b1.reshape(F, 1, U1),
                       W2, b2.reshape(F, 1, U2),
                       W3, b3.reshape(F, 1, DIM), codebook)
    table = codebook.T
    quant = _gather_rows_kernel()(table, idx.reshape(NW, NGCH, GCH))
    return _decode_call(quant.reshape(F, B, DIM),
                        W4, b4.reshape(F, 1, U2), W5, b5.reshape(F, 1, U1),
                        W6, b6.reshape(F, 1, FTS))


# speedup vs baseline: 1.3004x; 1.1338x over previous
import functools

import jax
import jax.numpy as jnp
from jax import lax
from jax.experimental import pallas as pl
from jax.experimental.pallas import tpu as pltpu
from jax.experimental.pallas import tpu_sc as plsc

F = 64
B = 1024
FTS = 128
U1 = 256
U2 = 128
DIM = 32
EMB = 8192

CHUNK = 1024
NCHUNK = EMB // CHUNK

NC = 2
NS = 16
NW = NC * NS
BPW = (F * B) // NW
GCH = 128
NGCH = BPW // GCH


def _encode_body(x_hbm, w1_ref, b1_ref, w2_ref, b2_ref, w3_ref, b3_ref,
                 cb_ref, idx_ref, xbuf, sem):
    f = pl.program_id(0)
    p = f % 2

    @pl.when(f == 0)
    def _():
        pltpu.make_async_copy(x_hbm.at[:, 0, :], xbuf.at[0], sem.at[0]).start()

    @pl.when(f + 1 < F)
    def _():
        pltpu.make_async_copy(x_hbm.at[:, f + 1, :], xbuf.at[1 - p],
                              sem.at[1 - p]).start()

    pltpu.make_async_copy(x_hbm.at[:, f, :], xbuf.at[p], sem.at[p]).wait()
    x = xbuf[p]
    h = jnp.maximum(
        jnp.dot(x, w1_ref[0], preferred_element_type=jnp.float32)
        + b1_ref[0], 0.0)
    h = jnp.maximum(
        jnp.dot(h, w2_ref[0], preferred_element_type=jnp.float32)
        + b2_ref[0], 0.0)
    z = jnp.maximum(
        jnp.dot(h, w3_ref[0], preferred_element_type=jnp.float32)
        + b3_ref[0], 0.0)
    z_aug = jnp.concatenate(
        [z, jnp.full((B, 1), -1.0, jnp.float32),
         jnp.zeros((B, 7), jnp.float32)], axis=1)
    runbest = jnp.full((B, 128), -jnp.inf, dtype=jnp.float32)
    runtile = jnp.zeros((B, 128), dtype=jnp.float32)
    ntile = CHUNK // 128
    for c in range(NCHUNK):
        cb = cb_ref[:, c * CHUNK:(c + 1) * CHUNK]
        csq2 = 0.5 * jnp.sum(cb * cb, axis=0, keepdims=True)
        cb_aug = jnp.concatenate(
            [cb, csq2, jnp.zeros((7, CHUNK), jnp.float32)], axis=0)
        s2 = jnp.dot(z_aug, cb_aug, preferred_element_type=jnp.float32)
        for t in range(ntile):
            st = s2[:, t * 128:(t + 1) * 128]
            upd = st > runbest
            runbest = jnp.where(upd, st, runbest)
            runtile = jnp.where(upd, float(c * ntile + t), runtile)
    best = jnp.max(runbest, axis=1, keepdims=True)
    lane = lax.broadcasted_iota(jnp.int32, (B, 128), 1).astype(jnp.float32)
    colg = runtile * 128.0 + lane
    cand = jnp.min(jnp.where(runbest >= best, colg, 3.0e7),
                   axis=1, keepdims=True)
    idx_ref[...] = cand.astype(jnp.int32)


def _decode_body(q_ref, w4_ref, b4_ref, w5_ref, b5_ref, w6_ref, b6_ref,
                 o_hbm, ybuf, sem):
    f = pl.program_id(0)
    p = f % 2

    @pl.when(f >= 2)
    def _():
        pltpu.make_async_copy(ybuf.at[p], o_hbm.at[:, f - 2, :],
                              sem.at[p]).wait()

    q = q_ref[0]
    h = jnp.maximum(
        jnp.dot(q, w4_ref[0], preferred_element_type=jnp.float32)
        + b4_ref[0], 0.0)
    h = jnp.maximum(
        jnp.dot(h, w5_ref[0], preferred_element_type=jnp.float32)
        + b5_ref[0], 0.0)
    y = jnp.dot(h, w6_ref[0], preferred_element_type=jnp.float32) + b6_ref[0]
    ybuf[p] = 1.0 / (1.0 + jnp.exp(-y))
    pltpu.make_async_copy(ybuf.at[p], o_hbm.at[:, f, :], sem.at[p]).start()

    @pl.when(f == F - 1)
    def _():
        pltpu.make_async_copy(ybuf.at[1 - p], o_hbm.at[:, f - 1, :],
                              sem.at[1 - p]).wait()
        pltpu.make_async_copy(ybuf.at[p], o_hbm.at[:, f, :],
                              sem.at[p]).wait()


@functools.cache
def _gather_rows_kernel():
    mesh = plsc.VectorSubcoreMesh(core_axis_name="c", subcore_axis_name="s",
                                  num_cores=NC)

    @functools.partial(
        pl.kernel,
        mesh=mesh,
        out_type=jax.ShapeDtypeStruct((F * B, DIM), jnp.float32),
        scratch_types=[
            pltpu.VMEM((NGCH, GCH), jnp.int32),
            pltpu.VMEM((BPW, DIM), jnp.float32),
            pltpu.SemaphoreType.DMA,
        ],
        compiler_params=pltpu.CompilerParams(use_tc_tiling_on_sc=False),
    )
    def _gather_rows(table_hbm, idx_hbm, out_hbm, idx_v, rows_v, sem):
        wid = lax.axis_index("s") * NC + lax.axis_index("c")
        pltpu.sync_copy(idx_hbm.at[wid], idx_v)
        copies = [
            pltpu.async_copy(table_hbm.at[idx_v.at[j]],
                             rows_v.at[pl.ds(j * GCH, GCH)], sem)
            for j in range(NGCH)
        ]
        for cp in copies:
            cp.wait()
        pltpu.sync_copy(rows_v, out_hbm.at[pl.ds(wid * BPW, BPW)])

    return _gather_rows


def _encode_call(x3d, W1, b1r, W2, b2r, W3, b3r, codebook, interpret=False):
    return pl.pallas_call(
        _encode_body,
        grid=(F,),
        in_specs=[
            pl.BlockSpec(memory_space=pl.ANY),
            pl.BlockSpec((1, FTS, U1), lambda f: (f, 0, 0)),
            pl.BlockSpec((1, 1, U1), lambda f: (f, 0, 0)),
            pl.BlockSpec((1, U1, U2), lambda f: (f, 0, 0)),
            pl.BlockSpec((1, 1, U2), lambda f: (f, 0, 0)),
            pl.BlockSpec((1, U2, DIM), lambda f: (f, 0, 0)),
            pl.BlockSpec((1, 1, DIM), lambda f: (f, 0, 0)),
            pl.BlockSpec((DIM, EMB), lambda f: (0, 0)),
        ],
        out_specs=pl.BlockSpec((B, 1), lambda f: (f, 0)),
        out_shape=jax.ShapeDtypeStruct((F * B, 1), jnp.int32),
        scratch_shapes=[
            pltpu.VMEM((2, B, FTS), jnp.float32),
            pltpu.SemaphoreType.DMA((2,)),
        ],
        compiler_params=pltpu.CompilerParams(
            dimension_semantics=("arbitrary",)),
        interpret=interpret,
    )(x3d, W1, b1r, W2, b2r, W3, b3r, codebook)


def _decode_call(q3, W4, b4r, W5, b5r, W6, b6r, interpret=False):
    return pl.pallas_call(
        _decode_body,
        grid=(F,),
        in_specs=[
            pl.BlockSpec((1, B, DIM), lambda f: (f, 0, 0)),
            pl.BlockSpec((1, DIM, U2), lambda f: (f, 0, 0)),
            pl.BlockSpec((1, 1, U2), lambda f: (f, 0, 0)),
            pl.BlockSpec((1, U2, U1), lambda f: (f, 0, 0)),
            pl.BlockSpec((1, 1, U1), lambda f: (f, 0, 0)),
            pl.BlockSpec((1, U1, FTS), lambda f: (f, 0, 0)),
            pl.BlockSpec((1, 1, FTS), lambda f: (f, 0, 0)),
        ],
        out_specs=pl.BlockSpec(memory_space=pl.ANY),
        out_shape=jax.ShapeDtypeStruct((B, F, FTS), jnp.float32),
        scratch_shapes=[
            pltpu.VMEM((2, B, FTS), jnp.float32),
            pltpu.SemaphoreType.DMA((2,)),
        ],
        compiler_params=pltpu.CompilerParams(
            dimension_semantics=("arbitrary",)),
        interpret=interpret,
    )(q3, W4, b4r, W5, b5r, W6, b6r)


def kernel(inputs, W1, b1, W2, b2, W3, b3, W4, b4, W5, b5, W6, b6, codebook):
    idx = _encode_call(inputs, W1, b1.reshape(F, 1, U1),
                       W2, b2.reshape(F, 1, U2),
                       W3, b3.reshape(F, 1, DIM), codebook)
    table = codebook.T
    quant = _gather_rows_kernel()(table, idx.reshape(NW, NGCH, GCH))
    return _decode_call(quant.reshape(F, B, DIM),
                        W4, b4.reshape(F, 1, U2), W5, b5.reshape(F, 1, U1),
                        W6, b6.reshape(F, 1, FTS))
